# baseline (speedup 1.0000x reference)
import functools

import jax
import jax.numpy as jnp
from jax import lax
from jax.experimental import pallas as pl
from jax.experimental.pallas import tpu as pltpu
from jax.experimental.pallas import tpu_sc as plsc

ROWS, COLS = 1_000_000, 64
NIDX = 16_384
NC, NS = 2, 16
NW = NC * NS
SLAB = ROWS // NW
IDX_CHUNK = 128
NGROUPS = NIDX // 16
CBUF = NIDX + 2 * IDX_CHUNK

_MESH = plsc.VectorSubcoreMesh(
    core_axis_name="c", subcore_axis_name="s", num_cores=NC, num_subcores=NS
)


@functools.partial(
    pl.kernel,
    out_type=jax.ShapeDtypeStruct((ROWS, COLS), jnp.float32),
    mesh=_MESH,
    compiler_params=pltpu.CompilerParams(
        use_tc_tiling_on_sc=False, needs_layout_passes=False
    ),
    scratch_types=[
        pltpu.VMEM((NIDX,), jnp.int32),
        pltpu.VMEM((CBUF // IDX_CHUNK, IDX_CHUNK), jnp.int32),
        pltpu.VMEM((IDX_CHUNK, COLS), jnp.float32),
        pltpu.VMEM((IDX_CHUNK, COLS), jnp.float32),
        pltpu.VMEM((1, COLS), jnp.float32),
        pltpu.SemaphoreType.DMA,
        pltpu.SemaphoreType.DMA,
    ],
)
def _sc_scatter_zero(inp_hbm, idx_hbm, out_hbm, idx_v, cbuf, zeros_v, tail_v,
                     row_v, copy_sem, scat_sem):
    wid = lax.axis_index("s") * NC + lax.axis_index("c")
    lo = wid * SLAB

    slab_cp = pltpu.make_async_copy(
        inp_hbm.at[pl.ds(lo, SLAB)], out_hbm.at[pl.ds(lo, SLAB)], copy_sem
    )
    slab_cp.start()

    pltpu.sync_copy(idx_hbm, idx_v)

    def scan_body(g, carry):
        cursor, hits = carry
        v = idx_v[pl.ds(g * 16, 16)]
        own = (v >= lo) & (v < lo + SLAB)
        csum = jnp.cumsum(jnp.where(own, 1, 0))
        pos = jnp.where(own, cursor + csum - 1, CBUF - IDX_CHUNK)
        plsc.store_scatter(cbuf, [pos // IDX_CHUNK, pos % IDX_CHUNK], v,
                           mask=own)
        cursor = cursor + jnp.max(csum)
        hits = hits + jnp.sum(jnp.where(v == lo, 1, 0))
        return cursor, hits

    cursor, hits = lax.fori_loop(0, NGROUPS, scan_body, (0, 0))

    sent = jnp.full((16,), lo, jnp.int32)
    for k in range(IDX_CHUNK // 16):
        pos = cursor + k * 16 + lax.iota(jnp.int32, 16)
        plsc.store_scatter(cbuf, [pos // IDX_CHUNK, pos % IDX_CHUNK], sent)
    n_chunks = (cursor + IDX_CHUNK - 1) // IDX_CHUNK

    zero16 = jnp.zeros((16,), jnp.float32)

    @pl.loop(0, IDX_CHUNK)
    def _(i):
        for c in range(COLS // 16):
            zeros_v[i, pl.ds(c * 16, 16)] = zero16

    pltpu.sync_copy(inp_hbm.at[pl.ds(lo, 1)], row_v)

    @pl.when(hits > 0)
    def _():
        for c in range(COLS // 16):
            row_v[0, pl.ds(c * 16, 16)] = zero16

    tail_cnt = cursor - (n_chunks - 1) * IDX_CHUNK

    @pl.loop(0, IDX_CHUNK)
    def _(i):
        pad = i >= tail_cnt
        for c in range(COLS // 16):
            rv = row_v[0, pl.ds(c * 16, 16)]
            tail_v[i, pl.ds(c * 16, 16)] = jnp.where(pad, rv, zero16)

    slab_cp.wait()

    @pl.loop(0, n_chunks)
    def _(j):
        @pl.when(j < n_chunks - 1)
        def _():
            cp = pltpu.make_async_copy(zeros_v, out_hbm.at[cbuf.at[j]],
                                       scat_sem)
            cp.start()
            cp.wait()

        @pl.when(j == n_chunks - 1)
        def _():
            cp = pltpu.make_async_copy(tail_v, out_hbm.at[cbuf.at[j]],
                                       scat_sem)
            cp.start()
            cp.wait()


def kernel(inp, indices):
    return _sc_scatter_zero(inp, indices)


# speedup vs baseline: 40.9635x; 40.9635x over previous
import functools

import jax
import jax.numpy as jnp
from jax import lax
from jax.experimental import pallas as pl
from jax.experimental.pallas import tpu as pltpu
from jax.experimental.pallas import tpu_sc as plsc

ROWS, COLS = 1_000_000, 64
NIDX = 16_384
NC, NS = 2, 16
NW = NC * NS
NGROUPS = NIDX // 16

BLK = 8192
NBLK = (ROWS + BLK - 1) // BLK
MPAD = NBLK * BLK
SEG = MPAD // NW

_MESH = plsc.VectorSubcoreMesh(
    core_axis_name="c", subcore_axis_name="s", num_cores=NC, num_subcores=NS
)


@functools.partial(
    pl.kernel,
    out_type=jax.ShapeDtypeStruct((MPAD,), jnp.float32),
    mesh=_MESH,
    compiler_params=pltpu.CompilerParams(
        use_tc_tiling_on_sc=False, needs_layout_passes=False
    ),
    scratch_types=[
        pltpu.VMEM((NIDX,), jnp.int32),
        pltpu.VMEM((SEG + 16,), jnp.float32),
    ],
)
def _sc_mask(idx_hbm, mask_hbm, idx_v, seg_v):
    wid = lax.axis_index("s") * NC + lax.axis_index("c")
    lo = wid * SEG

    pltpu.sync_copy(idx_hbm, idx_v)

    one16 = jnp.ones((16,), jnp.float32)

    @pl.loop(0, SEG // 16)
    def _(i):
        seg_v[pl.ds(i * 16, 16)] = one16

    zero16 = jnp.zeros((16,), jnp.float32)

    @pl.loop(0, NGROUPS)
    def _(g):
        v = idx_v[pl.ds(g * 16, 16)]
        own = (v >= lo) & (v < lo + SEG)
        plsc.store_scatter(seg_v, [jnp.where(own, v - lo, SEG)], zero16,
                           mask=own)

    pltpu.sync_copy(seg_v.at[pl.ds(0, SEG)], mask_hbm.at[pl.ds(lo, SEG)])


def _tc_body(mask_ref, x_ref, o_ref):
    o_ref[...] = jnp.where(mask_ref[0] == 0.0, 0.0, x_ref[...])


def _tc_apply(inp_t, mask):
    mask3 = mask.reshape(NBLK, 1, BLK)
    return pl.pallas_call(
        _tc_body,
        grid=(NBLK,),
        in_specs=[
            pl.BlockSpec((1, 1, BLK), lambda j: (j, 0, 0)),
            pl.BlockSpec((COLS, BLK), lambda j: (0, j)),
        ],
        out_specs=pl.BlockSpec((COLS, BLK), lambda j: (0, j)),
        out_shape=jax.ShapeDtypeStruct((COLS, ROWS), jnp.float32),
    )(mask3, inp_t)


def kernel(inp, indices):
    mask = _sc_mask(indices)
    out_t = _tc_apply(inp.T, mask)
    return out_t.T


# speedup vs baseline: 43.9973x; 1.0741x over previous
import functools

import jax
import jax.numpy as jnp
from jax import lax
from jax.experimental import pallas as pl
from jax.experimental.pallas import tpu as pltpu
from jax.experimental.pallas import tpu_sc as plsc

ROWS, COLS = 1_000_000, 64
NIDX = 16_384
NC, NS = 2, 16
NW = NC * NS
NGROUPS = NIDX // 16

BLK = 16384
NBLK = (ROWS + BLK - 1) // BLK
MPAD = NBLK * BLK
SEG = MPAD // NW

_MESH = plsc.VectorSubcoreMesh(
    core_axis_name="c", subcore_axis_name="s", num_cores=NC, num_subcores=NS
)


@functools.partial(
    pl.kernel,
    out_type=jax.ShapeDtypeStruct((MPAD,), jnp.float32),
    mesh=_MESH,
    compiler_params=pltpu.CompilerParams(
        use_tc_tiling_on_sc=False, needs_layout_passes=False
    ),
    scratch_types=[
        pltpu.VMEM((NIDX,), jnp.int32),
        pltpu.VMEM((SEG + 16,), jnp.float32),
    ],
)
def _sc_mask(idx_hbm, mask_hbm, idx_v, seg_v):
    wid = lax.axis_index("s") * NC + lax.axis_index("c")
    lo = wid * SEG

    pltpu.sync_copy(idx_hbm, idx_v)

    one16 = jnp.ones((16,), jnp.float32)

    @pl.loop(0, SEG // 16)
    def _(i):
        seg_v[pl.ds(i * 16, 16)] = one16

    zero16 = jnp.zeros((16,), jnp.float32)

    @pl.loop(0, NGROUPS)
    def _(g):
        v = idx_v[pl.ds(g * 16, 16)]
        own = (v >= lo) & (v < lo + SEG)
        plsc.store_scatter(seg_v, [jnp.where(own, v - lo, SEG)], zero16,
                           mask=own)

    pltpu.sync_copy(seg_v.at[pl.ds(0, SEG)], mask_hbm.at[pl.ds(lo, SEG)])


def _tc_body(mask_ref, x_ref, o_ref):
    o_ref[...] = jnp.where(mask_ref[0] == 0.0, 0.0, x_ref[...])


def _tc_apply(inp_t, mask):
    mask3 = mask.reshape(NBLK, 1, BLK)
    return pl.pallas_call(
        _tc_body,
        grid=(NBLK,),
        in_specs=[
            pl.BlockSpec((1, 1, BLK), lambda j: (j, 0, 0)),
            pl.BlockSpec((COLS, BLK), lambda j: (0, j)),
        ],
        out_specs=pl.BlockSpec((COLS, BLK), lambda j: (0, j)),
        out_shape=jax.ShapeDtypeStruct((COLS, ROWS), jnp.float32),
    )(mask3, inp_t)


def kernel(inp, indices):
    mask = _sc_mask(indices)
    out_t = _tc_apply(inp.T, mask)
    return out_t.T


# speedup vs baseline: 44.8336x; 1.0190x over previous
import functools

import jax
import jax.numpy as jnp
from jax import lax
from jax.experimental import pallas as pl
from jax.experimental.pallas import tpu as pltpu
from jax.experimental.pallas import tpu_sc as plsc

ROWS, COLS = 1_000_000, 64
NIDX = 16_384
NC, NS = 2, 16
NW = NC * NS
NGROUPS = NIDX // 16

BLK = 32768
NBLK = (ROWS + BLK - 1) // BLK
MPAD = NBLK * BLK
SEG = MPAD // NW

_MESH = plsc.VectorSubcoreMesh(
    core_axis_name="c", subcore_axis_name="s", num_cores=NC, num_subcores=NS
)


@functools.partial(
    pl.kernel,
    out_type=jax.ShapeDtypeStruct((MPAD,), jnp.float32),
    mesh=_MESH,
    compiler_params=pltpu.CompilerParams(
        use_tc_tiling_on_sc=False, needs_layout_passes=False
    ),
    scratch_types=[
        pltpu.VMEM((NIDX,), jnp.int32),
        pltpu.VMEM((SEG + 16,), jnp.float32),
    ],
)
def _sc_mask(idx_hbm, mask_hbm, idx_v, seg_v):
    wid = lax.axis_index("s") * NC + lax.axis_index("c")
    lo = wid * SEG

    pltpu.sync_copy(idx_hbm, idx_v)

    one16 = jnp.ones((16,), jnp.float32)

    @pl.loop(0, SEG // 16)
    def _(i):
        seg_v[pl.ds(i * 16, 16)] = one16

    zero16 = jnp.zeros((16,), jnp.float32)

    @pl.loop(0, NGROUPS)
    def _(g):
        v = idx_v[pl.ds(g * 16, 16)]
        own = (v >= lo) & (v < lo + SEG)
        plsc.store_scatter(seg_v, [jnp.where(own, v - lo, SEG)], zero16,
                           mask=own)

    pltpu.sync_copy(seg_v.at[pl.ds(0, SEG)], mask_hbm.at[pl.ds(lo, SEG)])


def _tc_body(mask_ref, x_ref, o_ref):
    o_ref[...] = jnp.where(mask_ref[0] == 0.0, 0.0, x_ref[...])


def _tc_apply(inp_t, mask):
    mask3 = mask.reshape(NBLK, 1, BLK)
    return pl.pallas_call(
        _tc_body,
        grid=(NBLK,),
        in_specs=[
            pl.BlockSpec((1, 1, BLK), lambda j: (j, 0, 0)),
            pl.BlockSpec((COLS, BLK), lambda j: (0, j)),
        ],
        out_specs=pl.BlockSpec((COLS, BLK), lambda j: (0, j)),
        out_shape=jax.ShapeDtypeStruct((COLS, ROWS), jnp.float32),
    )(mask3, inp_t)


def kernel(inp, indices):
    mask = _sc_mask(indices)
    out_t = _tc_apply(inp.T, mask)
    return out_t.T


# speedup vs baseline: 44.9933x; 1.0036x over previous
import functools

import jax
import jax.numpy as jnp
from jax import lax
from jax.experimental import pallas as pl
from jax.experimental.pallas import tpu as pltpu
from jax.experimental.pallas import tpu_sc as plsc

ROWS, COLS = 1_000_000, 64
NIDX = 16_384
NC, NS = 2, 16
NW = NC * NS
NGROUPS = NIDX // 16

BLK = 49152
NBLK = (ROWS + BLK - 1) // BLK
MPAD = NBLK * BLK
SEG = MPAD // NW

_MESH = plsc.VectorSubcoreMesh(
    core_axis_name="c", subcore_axis_name="s", num_cores=NC, num_subcores=NS
)


@functools.partial(
    pl.kernel,
    out_type=jax.ShapeDtypeStruct((MPAD,), jnp.float32),
    mesh=_MESH,
    compiler_params=pltpu.CompilerParams(
        use_tc_tiling_on_sc=False, needs_layout_passes=False
    ),
    scratch_types=[
        pltpu.VMEM((NIDX,), jnp.int32),
        pltpu.VMEM((SEG + 16,), jnp.float32),
    ],
)
def _sc_mask(idx_hbm, mask_hbm, idx_v, seg_v):
    wid = lax.axis_index("s") * NC + lax.axis_index("c")
    lo = wid * SEG

    pltpu.sync_copy(idx_hbm, idx_v)

    one16 = jnp.ones((16,), jnp.float32)

    @pl.loop(0, SEG // 16)
    def _(i):
        seg_v[pl.ds(i * 16, 16)] = one16

    zero16 = jnp.zeros((16,), jnp.float32)

    @pl.loop(0, NGROUPS)
    def _(g):
        v = idx_v[pl.ds(g * 16, 16)]
        own = (v >= lo) & (v < lo + SEG)
        plsc.store_scatter(seg_v, [jnp.where(own, v - lo, SEG)], zero16,
                           mask=own)

    pltpu.sync_copy(seg_v.at[pl.ds(0, SEG)], mask_hbm.at[pl.ds(lo, SEG)])


def _tc_body(mask_ref, x_ref, o_ref):
    o_ref[...] = jnp.where(mask_ref[0] == 0.0, 0.0, x_ref[...])


def _tc_apply(inp_t, mask):
    mask3 = mask.reshape(NBLK, 1, BLK)
    return pl.pallas_call(
        _tc_body,
        grid=(NBLK,),
        in_specs=[
            pl.BlockSpec((1, 1, BLK), lambda j: (j, 0, 0)),
            pl.BlockSpec((COLS, BLK), lambda j: (0, j)),
        ],
        out_specs=pl.BlockSpec((COLS, BLK), lambda j: (0, j)),
        out_shape=jax.ShapeDtypeStruct((COLS, ROWS), jnp.float32),
    )(mask3, inp_t)


def kernel(inp, indices):
    mask = _sc_mask(indices)
    out_t = _tc_apply(inp.T, mask)
    return out_t.T
